# baseline (speedup 1.0000x reference)
import jax
import jax.numpy as jnp
from jax import lax
from jax.experimental import pallas as pl
from jax.experimental.pallas import tpu as pltpu
from jax.experimental.pallas import tpu_sc as plsc

_B = 16384
_D = 32
_NC = 2
_NS = 16
_NW = _NC * _NS
_BPW = _B // _NW
_CH = 128
_NCHUNK = _BPW // _CH
_L = 16
_N = 1000001
_S = 1000008
_WAVE_LAG = 4


def _cf_body(uidx_hbm, midx_hbm, uf_hbm, ub_hbm, mf_hbm, mb_hbm, out_hbm,
             uidx_v, midx_v, uf_g, mf_g, ub_g, mb_g, out_v, sem):
    wid = lax.axis_index("s") * _NC + lax.axis_index("c")
    base = wid * _BPW

    for j in range(_NCHUNK):
        pltpu.sync_copy(uidx_hbm.at[pl.ds(base + j * _CH, _CH)], uidx_v.at[j])
        pltpu.sync_copy(midx_hbm.at[pl.ds(base + j * _CH, _CH)], midx_v.at[j])

    waves = []

    def fire(wave):
        waves.append(wave)
        if len(waves) > _WAVE_LAG:
            for cp in waves.pop(0):
                cp.wait()

    fire([pltpu.async_copy(ub_hbm.at[uidx_v.at[j]], ub_g.at[j], sem)
          for j in range(_NCHUNK)]
         + [pltpu.async_copy(mb_hbm.at[midx_v.at[j]], mb_g.at[j], sem)
            for j in range(_NCHUNK)])
    for d in range(_D):
        wave = []
        for j in range(_NCHUNK):
            row = d * _NCHUNK + j
            wave.append(pltpu.async_copy(
                uf_hbm.at[pl.ds(d * _S, _S)].at[uidx_v.at[j]],
                uf_g.at[row], sem))
            wave.append(pltpu.async_copy(
                mf_hbm.at[pl.ds(d * _S, _S)].at[midx_v.at[j]],
                mf_g.at[row], sem))
        fire(wave)
    for wave in waves:
        for cp in wave:
            cp.wait()

    for blk in range(_BPW // _L):
        j, o = blk // (_CH // _L), (blk % (_CH // _L)) * _L
        accs = [
            ub_g[j, pl.ds(o, _L)] + mb_g[j, pl.ds(o, _L)],
            jnp.zeros((_L,), jnp.float32),
            jnp.zeros((_L,), jnp.float32),
            jnp.zeros((_L,), jnp.float32),
        ]
        for d in range(_D):
            row = d * _NCHUNK + j
            accs[d % 4] = accs[d % 4] + (uf_g[row, pl.ds(o, _L)]
                                         * mf_g[row, pl.ds(o, _L)])
        out_v[pl.ds(blk * _L, _L)] = (
            (accs[0] + accs[1]) + (accs[2] + accs[3]))

    pltpu.sync_copy(out_v, out_hbm.at[pl.ds(base, _BPW)])


@jax.jit
def _cf_call(u_idx, m_idx, uf_l, ub_l, mf_l, mb_l):
    mesh = plsc.VectorSubcoreMesh(core_axis_name="c", subcore_axis_name="s",
                                  num_cores=_NC, num_subcores=_NS)
    return pl.kernel(
        _cf_body,
        out_type=jax.ShapeDtypeStruct((_B,), jnp.float32),
        mesh=mesh,
        scratch_types=[
            pltpu.VMEM((_NCHUNK, _CH), jnp.int32),
            pltpu.VMEM((_NCHUNK, _CH), jnp.int32),
            pltpu.VMEM((_D * _NCHUNK, _CH), jnp.float32),
            pltpu.VMEM((_D * _NCHUNK, _CH), jnp.float32),
            pltpu.VMEM((_NCHUNK, _CH), jnp.float32),
            pltpu.VMEM((_NCHUNK, _CH), jnp.float32),
            pltpu.VMEM((_BPW,), jnp.float32),
            pltpu.SemaphoreType.DMA,
        ],
        compiler_params=pltpu.CompilerParams(
            needs_layout_passes=False, use_tc_tiling_on_sc=False),
    )(u_idx, m_idx, uf_l, ub_l, mf_l, mb_l)


def kernel(x, user_factors, user_bias, movie_factors, movie_bias):
    u_idx = x[:, 0]
    m_idx = x[:, 1]
    uf_l = jnp.pad(user_factors.T, ((0, 0), (0, _S - _N))).reshape(-1)
    mf_l = jnp.pad(movie_factors.T, ((0, 0), (0, _S - _N))).reshape(-1)
    return _cf_call(u_idx, m_idx, uf_l, user_bias[:, 0],
                    mf_l, movie_bias[:, 0])


# speedup vs baseline: 16.9075x; 16.9075x over previous
import jax
import jax.numpy as jnp
from jax import lax
from jax.experimental import pallas as pl
from jax.experimental.pallas import tpu as pltpu
from jax.experimental.pallas import tpu_sc as plsc

_B = 16384
_D = 32
_NC = 2
_NS = 16
_NW = _NC * _NS
_BPW = _B // _NW
_CH = 128
_NCHUNK = _BPW // _CH
_L = 16
_NB = _D // 8
_SLOTS = 8


def _cf_body(uidx_hbm, midx_hbm, uf_hbm, ub_hbm, mf_hbm, mb_hbm, out_hbm,
             uidx_v, midx_v, stg_u, stg_m, colu, colm, ub_g, mb_g, out_v,
             sem, bsem):
    wid = lax.axis_index("s") * _NC + lax.axis_index("c")
    base = wid * _BPW

    for j in range(_NCHUNK):
        pltpu.sync_copy(uidx_hbm.at[pl.ds(base + j * _CH, _CH)], uidx_v.at[j])
        pltpu.sync_copy(midx_hbm.at[pl.ds(base + j * _CH, _CH)], midx_v.at[j])

    bias_cps = [pltpu.async_copy(ub_hbm.at[uidx_v.at[j]], ub_g.at[j], bsem)
                for j in range(_NCHUNK)]
    bias_cps += [pltpu.async_copy(mb_hbm.at[midx_v.at[j]], mb_g.at[j], bsem)
                 for j in range(_NCHUNK)]

    lanes = jnp.arange(_L, dtype=jnp.int32)

    def fire(tbl, stg, slot, col_idx):
        c = pl.multiple_of((col_idx // _CH) * _CH, _CH)
        return [pltpu.async_copy(
            tbl.at[pl.ds(b * 8, 8), pl.ds(c, _CH)],
            stg.at[slot, pl.ds(b * 8, 8), :], sem) for b in range(_NB)]

    def extract(stg, col, cps, slot, col_idx, r):
        for cp in cps:
            cp.wait()
        ulane = col_idx - (col_idx // _CH) * _CH
        slotv = jnp.full((_L,), slot, jnp.int32)
        for k in range(_D // _L):
            dv = lanes + (k * _L)
            src_lane = jnp.full((_L,), ulane, jnp.int32)
            vals = plsc.load_gather(stg, [slotv, dv, src_lane])
            plsc.store_scatter(col, [(dv * _BPW) + r], vals)

    def group(g, carry):
        j = g // (_CH // _L)
        off = (g - j * (_CH // _L)) * _L
        uvec = uidx_v[j, pl.ds(off, _L)]
        mvec = midx_v[j, pl.ds(off, _L)]
        pend = {}
        for lane in range(_L):
            slot = lane % _SLOTS
            if lane >= _SLOTS:
                pu, pm, pul, pml, pr = pend.pop(slot)
                extract(stg_u, colu, pu, slot, pul, pr)
                extract(stg_m, colm, pm, slot, pml, pr)
            r = g * _L + lane
            u = uvec[lane]
            m = mvec[lane]
            pend[slot] = (fire(uf_hbm, stg_u, slot, u),
                          fire(mf_hbm, stg_m, slot, m), u, m, r)
        for lane in range(_SLOTS):
            pu, pm, pul, pml, pr = pend.pop(lane)
            extract(stg_u, colu, pu, lane, pul, pr)
            extract(stg_m, colm, pm, lane, pml, pr)
        return carry

    lax.fori_loop(0, _BPW // _L, group, 0)

    for cp in bias_cps:
        cp.wait()

    def dot_block(blk, carry):
        j = blk // (_CH // _L)
        o = (blk - j * (_CH // _L)) * _L
        acc0 = ub_g[j, pl.ds(o, _L)] + mb_g[j, pl.ds(o, _L)]
        acc1 = jnp.zeros((_L,), jnp.float32)
        acc2 = jnp.zeros((_L,), jnp.float32)
        acc3 = jnp.zeros((_L,), jnp.float32)
        accs = [acc0, acc1, acc2, acc3]
        p = blk * _L
        for d in range(_D):
            accs[d % 4] = accs[d % 4] + (colu[pl.ds(d * _BPW + p, _L)]
                                         * colm[pl.ds(d * _BPW + p, _L)])
        out_v[pl.ds(p, _L)] = (accs[0] + accs[1]) + (accs[2] + accs[3])
        return carry

    lax.fori_loop(0, _BPW // _L, dot_block, 0)

    pltpu.sync_copy(out_v, out_hbm.at[pl.ds(base, _BPW)])


@jax.jit
def _cf_call(u_idx, m_idx, uf_t, ub_l, mf_t, mb_l):
    mesh = plsc.VectorSubcoreMesh(core_axis_name="c", subcore_axis_name="s",
                                  num_cores=_NC, num_subcores=_NS)
    return pl.kernel(
        _cf_body,
        out_type=jax.ShapeDtypeStruct((_B,), jnp.float32),
        mesh=mesh,
        scratch_types=[
            pltpu.VMEM((_NCHUNK, _CH), jnp.int32),
            pltpu.VMEM((_NCHUNK, _CH), jnp.int32),
            pltpu.VMEM((_SLOTS, _D, _CH), jnp.float32),
            pltpu.VMEM((_SLOTS, _D, _CH), jnp.float32),
            pltpu.VMEM((_D * _BPW,), jnp.float32),
            pltpu.VMEM((_D * _BPW,), jnp.float32),
            pltpu.VMEM((_NCHUNK, _CH), jnp.float32),
            pltpu.VMEM((_NCHUNK, _CH), jnp.float32),
            pltpu.VMEM((_BPW,), jnp.float32),
            pltpu.SemaphoreType.DMA,
            pltpu.SemaphoreType.DMA,
        ],
        compiler_params=pltpu.CompilerParams(
            needs_layout_passes=False, use_tc_tiling_on_sc=True),
    )(u_idx, m_idx, uf_t, ub_l, mf_t, mb_l)


def kernel(x, user_factors, user_bias, movie_factors, movie_bias):
    u_idx = x[:, 0]
    m_idx = x[:, 1]
    return _cf_call(u_idx, m_idx, user_factors.T, user_bias[:, 0],
                    movie_factors.T, movie_bias[:, 0])


# speedup vs baseline: 17.1264x; 1.0129x over previous
import jax
import jax.numpy as jnp
from jax import lax
from jax.experimental import pallas as pl
from jax.experimental.pallas import tpu as pltpu
from jax.experimental.pallas import tpu_sc as plsc

_B = 16384
_D = 32
_NC = 2
_NS = 16
_NW = _NC * _NS
_BPW = _B // _NW
_CH = 128
_NCHUNK = _BPW // _CH
_L = 16
_NB = _D // 8
_SLOTS = 8


def _cf_body(uidx_hbm, midx_hbm, uf_hbm, ub_hbm, mf_hbm, mb_hbm, out_hbm,
             uidx_v, midx_v, stg_u, stg_m, colu, colm, ub_g, mb_g, out_v,
             sem, bsem):
    wid = lax.axis_index("s") * _NC + lax.axis_index("c")
    base = wid * _BPW

    for j in range(_NCHUNK):
        pltpu.sync_copy(uidx_hbm.at[pl.ds(base + j * _CH, _CH)], uidx_v.at[j])
        pltpu.sync_copy(midx_hbm.at[pl.ds(base + j * _CH, _CH)], midx_v.at[j])

    bias_cps = [pltpu.async_copy(ub_hbm.at[uidx_v.at[j]], ub_g.at[j], bsem)
                for j in range(_NCHUNK)]
    bias_cps += [pltpu.async_copy(mb_hbm.at[midx_v.at[j]], mb_g.at[j], bsem)
                 for j in range(_NCHUNK)]

    lanes = jnp.arange(_L, dtype=jnp.int32)

    def fire(tbl, stg, slot, col_idx):
        c = pl.multiple_of((col_idx // _CH) * _CH, _CH)
        return [pltpu.async_copy(
            tbl.at[:, pl.ds(c, _CH)], stg.at[slot], sem)]

    def extract(stg, col, cps, slot, col_idx, r):
        for cp in cps:
            cp.wait()
        ulane = col_idx - (col_idx // _CH) * _CH
        slotv = jnp.full((_L,), slot, jnp.int32)
        for k in range(_D // _L):
            dv = lanes + (k * _L)
            src_lane = jnp.full((_L,), ulane, jnp.int32)
            vals = plsc.load_gather(stg, [slotv, dv, src_lane])
            plsc.store_scatter(col, [(dv * _BPW) + r], vals)

    def group(g, carry):
        j = g // (_CH // _L)
        off = (g - j * (_CH // _L)) * _L
        uvec = uidx_v[j, pl.ds(off, _L)]
        mvec = midx_v[j, pl.ds(off, _L)]
        pend = {}
        for lane in range(_L):
            slot = lane % _SLOTS
            if lane >= _SLOTS:
                pu, pm, pul, pml, pr = pend.pop(slot)
                extract(stg_u, colu, pu, slot, pul, pr)
                extract(stg_m, colm, pm, slot, pml, pr)
            r = g * _L + lane
            u = uvec[lane]
            m = mvec[lane]
            pend[slot] = (fire(uf_hbm, stg_u, slot, u),
                          fire(mf_hbm, stg_m, slot, m), u, m, r)
        for lane in range(_SLOTS):
            pu, pm, pul, pml, pr = pend.pop(lane)
            extract(stg_u, colu, pu, lane, pul, pr)
            extract(stg_m, colm, pm, lane, pml, pr)
        return carry

    lax.fori_loop(0, _BPW // _L, group, 0)

    for cp in bias_cps:
        cp.wait()

    def dot_block(blk, carry):
        j = blk // (_CH // _L)
        o = (blk - j * (_CH // _L)) * _L
        acc0 = ub_g[j, pl.ds(o, _L)] + mb_g[j, pl.ds(o, _L)]
        acc1 = jnp.zeros((_L,), jnp.float32)
        acc2 = jnp.zeros((_L,), jnp.float32)
        acc3 = jnp.zeros((_L,), jnp.float32)
        accs = [acc0, acc1, acc2, acc3]
        p = blk * _L
        for d in range(_D):
            accs[d % 4] = accs[d % 4] + (colu[pl.ds(d * _BPW + p, _L)]
                                         * colm[pl.ds(d * _BPW + p, _L)])
        out_v[pl.ds(p, _L)] = (accs[0] + accs[1]) + (accs[2] + accs[3])
        return carry

    lax.fori_loop(0, _BPW // _L, dot_block, 0)

    pltpu.sync_copy(out_v, out_hbm.at[pl.ds(base, _BPW)])


@jax.jit
def _cf_call(u_idx, m_idx, uf_t, ub_l, mf_t, mb_l):
    mesh = plsc.VectorSubcoreMesh(core_axis_name="c", subcore_axis_name="s",
                                  num_cores=_NC, num_subcores=_NS)
    return pl.kernel(
        _cf_body,
        out_type=jax.ShapeDtypeStruct((_B,), jnp.float32),
        mesh=mesh,
        scratch_types=[
            pltpu.VMEM((_NCHUNK, _CH), jnp.int32),
            pltpu.VMEM((_NCHUNK, _CH), jnp.int32),
            pltpu.VMEM((_SLOTS, _D, _CH), jnp.float32),
            pltpu.VMEM((_SLOTS, _D, _CH), jnp.float32),
            pltpu.VMEM((_D * _BPW,), jnp.float32),
            pltpu.VMEM((_D * _BPW,), jnp.float32),
            pltpu.VMEM((_NCHUNK, _CH), jnp.float32),
            pltpu.VMEM((_NCHUNK, _CH), jnp.float32),
            pltpu.VMEM((_BPW,), jnp.float32),
            pltpu.SemaphoreType.DMA,
            pltpu.SemaphoreType.DMA,
        ],
        compiler_params=pltpu.CompilerParams(
            needs_layout_passes=False, use_tc_tiling_on_sc=True),
    )(u_idx, m_idx, uf_t, ub_l, mf_t, mb_l)


def kernel(x, user_factors, user_bias, movie_factors, movie_bias):
    u_idx = x[:, 0]
    m_idx = x[:, 1]
    return _cf_call(u_idx, m_idx, user_factors.T, user_bias[:, 0],
                    movie_factors.T, movie_bias[:, 0])
